# baseline (speedup 1.0000x reference)
import jax
import jax.numpy as jnp
from jax.experimental import pallas as pl
from jax.experimental.pallas import tpu as pltpu

B_BOX = 2
C_CLS = 20
LAMBDA_COORD = 5.0
LAMBDA_NOOBJ = 0.5
N_ELEM = B_BOX * 5 + C_CLS
BATCH = 4096
S = 7
N_CELLS = BATCH * S * S

PACK = 4
W = PACK * N_ELEM
WP = 128
N4 = N_CELLS // PACK
GRID = 28
ROWS = N4 // GRID
CHUNK = 16


def _rl(x, k):
    return pltpu.roll(x, WP - k, 1)


def _chunk_loss(x, y):
    d = x - y
    d2 = d * d
    s1 = d2 + _rl(d2, 1)
    s2 = s1 + _rl(s1, 2)
    s4 = s2 + _rl(s2, 4)
    s8 = s4 + _rl(s4, 8)
    s_box0 = s2
    s_box1 = _rl(s2, 5)
    s_class = _rl(s8, 10) + _rl(s2, 26)
    c0 = _rl(d2, 4)
    c1 = _rl(d2, 9)

    x2, x5, x7 = _rl(x, 2), _rl(x, 5), _rl(x, 7)
    y2, y5, y7 = _rl(y, 2), _rl(y, 5), _rl(y, 7)

    def wh(ahi, bhi, alo, blo):
        return jnp.maximum(jnp.minimum(ahi, bhi) - jnp.maximum(alo, blo), 0.0)

    wh_c = wh(x2, y2, x, y)
    wh_a = wh(x7, y2, x5, y)
    wh_b = wh(x2, y7, x, y5)
    inter_c = wh_c * _rl(wh_c, 1)
    inter_a = wh_a * _rl(wh_a, 1)
    inter_b = wh_b * _rl(wh_b, 1)

    ex = x2 - x
    ap = ex * _rl(ex, 1)
    ey = y2 - y
    at = ey * _rl(ey, 1)
    ap5 = _rl(ap, 5)
    at5 = _rl(at, 5)

    iou_c = inter_c / (ap + at - inter_c)
    iou_a = inter_a / (ap5 + at - inter_a)
    iou_b = inter_b / (ap + at5 - inter_b)

    m0 = iou_a > iou_c
    m1 = _rl(iou_c, 5) > iou_b
    coordf = jnp.where(y5 > 0.0, 1.0, 0.0)
    nw = jnp.where(y5 == 0.0, LAMBDA_NOOBJ, 0.0)
    r0 = jnp.where(~m0 | ~m1, coordf, 0.0)
    r1 = jnp.where(m0 | m1, coordf, 0.0)

    return (
        LAMBDA_COORD * (r0 * s_box0 + r1 * s_box1)
        + r0 * c0 + r1 * c1
        + nw * (c0 + c1)
        + coordf * s_class
    )


def _block_body(p_ref, t_ref, o_ref):
    lane = jax.lax.broadcasted_iota(jnp.int32, (CHUNK, WP), 1)
    keep = (lane % N_ELEM == 0) & (lane < W)
    zpad = jnp.zeros((CHUNK, WP - W), jnp.float32)

    def step(i, acc):
        x = jnp.concatenate([p_ref[pl.ds(i * CHUNK, CHUNK), :], zpad], axis=1)
        y = jnp.concatenate([t_ref[pl.ds(i * CHUNK, CHUNK), :], zpad], axis=1)
        return acc + jnp.where(keep, _chunk_loss(x, y), 0.0)

    acc = jax.lax.fori_loop(
        0, ROWS // CHUNK, step, jnp.zeros((CHUNK, WP), jnp.float32)
    )
    partial = jnp.sum(acc).reshape(1, 1)

    @pl.when(pl.program_id(0) == 0)
    def _():
        o_ref[...] = jnp.zeros((1, 1), jnp.float32)

    o_ref[...] += partial


def kernel(pred_tensor, target_tensor):
    p = pred_tensor.reshape(N4, W)
    t = target_tensor.reshape(N4, W)
    out = pl.pallas_call(
        _block_body,
        grid=(GRID,),
        in_specs=[
            pl.BlockSpec((ROWS, W), lambda i: (i, 0)),
            pl.BlockSpec((ROWS, W), lambda i: (i, 0)),
        ],
        out_specs=pl.BlockSpec((1, 1), lambda i: (0, 0)),
        out_shape=jax.ShapeDtypeStruct((1, 1), jnp.float32),
    )(p, t)
    return out[0, 0]


# speedup vs baseline: 3.6020x; 3.6020x over previous
import jax
import jax.numpy as jnp
from jax.experimental import pallas as pl
from jax.experimental.pallas import tpu as pltpu

B_BOX = 2
C_CLS = 20
LAMBDA_COORD = 5.0
LAMBDA_NOOBJ = 0.5
N_ELEM = B_BOX * 5 + C_CLS
BATCH = 4096
S = 7
N_CELLS = BATCH * S * S

PACK = 4
W = PACK * N_ELEM
WP = 128
N4 = N_CELLS // PACK
GRID = 28
ROWS = N4 // GRID


def _rl(x, k):
    return pltpu.roll(x, WP - k, 1)


def _block_body(p_ref, t_ref, o_ref):
    zpad = jnp.zeros((ROWS, WP - W), jnp.float32)
    x = jnp.concatenate([p_ref[...], zpad], axis=1)
    y = jnp.concatenate([t_ref[...], zpad], axis=1)

    d = x - y
    d2 = d * d
    s1 = d2 + _rl(d2, 1)
    s2 = s1 + _rl(s1, 2)
    s4 = s2 + _rl(s2, 4)
    s8 = s4 + _rl(s4, 8)
    s_box0 = s2
    s_box1 = _rl(s2, 5)
    s_class = _rl(s8, 10) + _rl(s2, 26)
    c0 = _rl(d2, 4)
    c1 = _rl(d2, 9)

    x2, x5, x7 = _rl(x, 2), _rl(x, 5), _rl(x, 7)
    y2, y5, y7 = _rl(y, 2), _rl(y, 5), _rl(y, 7)

    def wh(ahi, bhi, alo, blo):
        return jnp.maximum(jnp.minimum(ahi, bhi) - jnp.maximum(alo, blo), 0.0)

    wh_c = wh(x2, y2, x, y)
    wh_a = wh(x7, y2, x5, y)
    wh_b = wh(x2, y7, x, y5)
    i_c = wh_c * _rl(wh_c, 1)
    i_a = wh_a * _rl(wh_a, 1)
    i_b = wh_b * _rl(wh_b, 1)

    ex = x2 - x
    ap = ex * _rl(ex, 1)
    ey = y2 - y
    at = ey * _rl(ey, 1)
    ap5 = _rl(ap, 5)
    at5 = _rl(at, 5)

    d_c = ap + at - i_c
    d_a = ap5 + at - i_a
    d_b = ap + at5 - i_b
    i_c5 = _rl(i_c, 5)
    d_c5 = _rl(d_c, 5)

    dl0 = i_a * d_c - i_c * d_a
    dl1 = i_c5 * d_b - i_b * d_c5
    m0 = jnp.where(d_a * d_c < 0.0, -dl0, dl0) > 0.0
    m1 = jnp.where(d_c5 * d_b < 0.0, -dl1, dl1) > 0.0

    coordf = jnp.where(y5 > 0.0, 1.0, 0.0)
    nw = jnp.where(y5 == 0.0, LAMBDA_NOOBJ, 0.0)
    r0 = jnp.where(~(m0 & m1), coordf, 0.0)
    r1 = jnp.where(m0 | m1, coordf, 0.0)

    per_cell = (
        LAMBDA_COORD * (r0 * s_box0 + r1 * s_box1)
        + r0 * c0 + r1 * c1
        + nw * (c0 + c1)
        + coordf * s_class
    )
    lane = jax.lax.broadcasted_iota(jnp.int32, (ROWS, WP), 1)
    masked = jnp.where((lane % N_ELEM == 0) & (lane < W), per_cell, 0.0)
    o_ref[...] = jnp.sum(masked).reshape(1, 1, 1)


def kernel(pred_tensor, target_tensor):
    p = pred_tensor.reshape(N4, W)
    t = target_tensor.reshape(N4, W)
    partials = pl.pallas_call(
        _block_body,
        grid=(GRID,),
        in_specs=[
            pl.BlockSpec((ROWS, W), lambda i: (i, 0)),
            pl.BlockSpec((ROWS, W), lambda i: (i, 0)),
        ],
        out_specs=pl.BlockSpec((1, 1, 1), lambda i: (i, 0, 0)),
        out_shape=jax.ShapeDtypeStruct((GRID, 1, 1), jnp.float32),
        compiler_params=pltpu.CompilerParams(
            dimension_semantics=("parallel",),
        ),
    )(p, t)
    return jnp.sum(partials)
